# baseline (speedup 1.0000x reference)
import functools
import numpy as np
import jax
from jax import lax
import jax.numpy as jnp
from jax.experimental import pallas as pl
from jax.experimental.pallas import tpu as pltpu
from jax.experimental.pallas import tpu_sc as plsc

L, H, D = 2048, 16, 128
BLKQ, BLKK = 128, 64
NQ, NK = L // BLKQ, L // BLKK
TOPK = max(1, int(0.1 * NK))
SCALE = 1.0 / np.sqrt(D)


def _scores_kernel(q_ref, k_ref, s_ref):
    qh = q_ref[...]
    kh = k_ref[...]
    q_pool = jnp.mean(qh.reshape(NQ, BLKQ, D), axis=1)
    k_pool = jnp.mean(kh.reshape(NK, BLKK, D), axis=1)
    s_ref[0] = jax.lax.dot_general(k_pool, q_pool, (((1,), (1,)), ((), ())),
                                   preferred_element_type=jnp.float32)


def _sc_topk_body(s_hbm, idx_hbm, s_v, o_v):
    wid = lax.axis_index("s") * 2 + lax.axis_index("c")

    @pl.when(wid < H)
    def _():
        pltpu.sync_copy(s_hbm.at[wid], s_v)
        svals = [s_v[k] for k in range(NK)]
        neg_inf = jnp.full((NQ,), -jnp.inf, jnp.float32)
        for j in range(TOPK):
            m = svals[0]
            for k in range(1, NK):
                m = jnp.maximum(m, svals[k])
            il = jnp.full((NQ,), NK, jnp.int32)
            for k in range(NK):
                il = jnp.minimum(il, jnp.where(svals[k] >= m, k, NK))
            o_v[j] = il
            hit = [il == k for k in range(NK)]
            svals = [jnp.where(hit[k], neg_inf, svals[k]) for k in range(NK)]
        pltpu.sync_copy(o_v, idx_hbm.at[wid])


def _attn_kernel(idx_ref, q_ref, k_ref, v_ref, o_ref):
    h = pl.program_id(0)

    def scores_for(qi):
        qb = (q_ref[qi * BLKQ:(qi + 1) * BLKQ, :] * SCALE).astype(jnp.bfloat16)
        ss = []
        vparts = []
        for j in range(TOPK):
            start = idx_ref[h, j, qi] * BLKK
            kj = k_ref[pl.ds(start, BLKK), :].astype(jnp.bfloat16)
            vparts.append(v_ref[pl.ds(start, BLKK), :].astype(jnp.bfloat16))
            ss.append(jax.lax.dot_general(qb, kj, (((1,), (1,)), ((), ())),
                                          preferred_element_type=jnp.float32))
        return ss, vparts

    def finish(qi, ss, vparts):
        ps = [jnp.exp(t) for t in ss]
        denom = (jnp.sum(ps[0], axis=1, keepdims=True)
                 + jnp.sum(ps[1], axis=1, keepdims=True)
                 + jnp.sum(ps[2], axis=1, keepdims=True))
        acc = jax.lax.dot(ps[0].astype(jnp.bfloat16), vparts[0],
                          preferred_element_type=jnp.float32)
        acc += jax.lax.dot(ps[1].astype(jnp.bfloat16), vparts[1],
                           preferred_element_type=jnp.float32)
        acc += jax.lax.dot(ps[2].astype(jnp.bfloat16), vparts[2],
                           preferred_element_type=jnp.float32)
        o_ref[qi * BLKQ:(qi + 1) * BLKQ, :] = acc / denom

    prev = scores_for(0)
    for qi in range(1, NQ):
        cur = scores_for(qi)
        finish(qi - 1, *prev)
        prev = cur
    finish(NQ - 1, *prev)


def kernel(q, k, v, W, b):
    qf = q.reshape(L, H * D)
    kf = k.reshape(L, H * D)
    vf = v.reshape(L, H * D)

    scores = pl.pallas_call(
        _scores_kernel,
        grid=(H,),
        in_specs=[
            pl.BlockSpec((L, D), lambda h: (0, h)),
            pl.BlockSpec((L, D), lambda h: (0, h)),
        ],
        out_specs=pl.BlockSpec((1, NK, NQ), lambda h: (h, 0, 0)),
        out_shape=jax.ShapeDtypeStruct((H, NK, NQ), jnp.float32),
    )(qf, kf)

    sc_topk = functools.partial(
        pl.kernel,
        mesh=plsc.VectorSubcoreMesh(core_axis_name="c", subcore_axis_name="s"),
        out_type=jax.ShapeDtypeStruct((H, TOPK, NQ), jnp.int32),
        scratch_types=[
            pltpu.VMEM((NK, NQ), jnp.float32),
            pltpu.VMEM((TOPK, NQ), jnp.int32),
        ],
    )(_sc_topk_body)
    idx_full = sc_topk(scores)

    grid_spec = pltpu.PrefetchScalarGridSpec(
        num_scalar_prefetch=1,
        grid=(H,),
        in_specs=[
            pl.BlockSpec((L, D), lambda h, idx_ref: (0, h)),
            pl.BlockSpec((L, D), lambda h, idx_ref: (0, h)),
            pl.BlockSpec((L, D), lambda h, idx_ref: (0, h)),
        ],
        out_specs=pl.BlockSpec((L, D), lambda h, idx_ref: (0, h)),
    )
    o = pl.pallas_call(
        _attn_kernel,
        grid_spec=grid_spec,
        out_shape=jax.ShapeDtypeStruct((L, H * D), jnp.float32),
    )(idx_full, qf, kf, vf)

    return o.reshape(q.shape)


# speedup vs baseline: 1.0386x; 1.0386x over previous
import functools
import numpy as np
import jax
from jax import lax
import jax.numpy as jnp
from jax.experimental import pallas as pl
from jax.experimental.pallas import tpu as pltpu
from jax.experimental.pallas import tpu_sc as plsc

L, H, D = 2048, 16, 128
BLKQ, BLKK = 128, 64
NQ, NK = L // BLKQ, L // BLKK
TOPK = max(1, int(0.1 * NK))
SCALE = 1.0 / np.sqrt(D)


def _scores_kernel(q_ref, k_ref, s_ref):
    qn = q_ref[0]
    kn = k_ref[0]
    q_pool = jnp.mean(qn.reshape(NQ, BLKQ, H, D), axis=1)
    k_pool = jnp.mean(kn.reshape(NK, BLKK, H, D), axis=1)
    for h in range(H):
        s_ref[h] = jax.lax.dot_general(
            k_pool[:, h, :], q_pool[:, h, :], (((1,), (1,)), ((), ())),
            preferred_element_type=jnp.float32)


def _sc_topk_body(s_hbm, idx_hbm, s_v, o_v):
    wid = lax.axis_index("s") * 2 + lax.axis_index("c")

    @pl.when(wid < H)
    def _():
        pltpu.sync_copy(s_hbm.at[wid], s_v)
        svals = [s_v[k] for k in range(NK)]
        neg_inf = jnp.full((NQ,), -jnp.inf, jnp.float32)
        for j in range(TOPK):
            m = svals[0]
            for k in range(1, NK):
                m = jnp.maximum(m, svals[k])
            il = jnp.full((NQ,), NK, jnp.int32)
            for k in range(NK):
                il = jnp.minimum(il, jnp.where(svals[k] >= m, k, NK))
            o_v[j] = il
            hit = [il == k for k in range(NK)]
            svals = [jnp.where(hit[k], neg_inf, svals[k]) for k in range(NK)]
        pltpu.sync_copy(o_v, idx_hbm.at[wid])


def _attn_kernel(idx_ref, q_ref, k_ref, v_ref, o_ref):
    h = pl.program_id(0)

    def scores_for(qi):
        qb = (q_ref[qi * BLKQ:(qi + 1) * BLKQ, :] * SCALE).astype(jnp.bfloat16)
        ss = []
        vparts = []
        for j in range(TOPK):
            start = idx_ref[h, j, qi] * BLKK
            kj = k_ref[pl.ds(start, BLKK), :].astype(jnp.bfloat16)
            vparts.append(v_ref[pl.ds(start, BLKK), :].astype(jnp.bfloat16))
            ss.append(jax.lax.dot_general(qb, kj, (((1,), (1,)), ((), ())),
                                          preferred_element_type=jnp.float32))
        return ss, vparts

    def finish(qi, ss, vparts):
        ps = [jnp.exp(t) for t in ss]
        denom = (jnp.sum(ps[0], axis=1, keepdims=True)
                 + jnp.sum(ps[1], axis=1, keepdims=True)
                 + jnp.sum(ps[2], axis=1, keepdims=True))
        acc = jax.lax.dot(ps[0].astype(jnp.bfloat16), vparts[0],
                          preferred_element_type=jnp.float32)
        acc += jax.lax.dot(ps[1].astype(jnp.bfloat16), vparts[1],
                           preferred_element_type=jnp.float32)
        acc += jax.lax.dot(ps[2].astype(jnp.bfloat16), vparts[2],
                           preferred_element_type=jnp.float32)
        o_ref[qi * BLKQ:(qi + 1) * BLKQ, :] = acc / denom

    prev = scores_for(0)
    for qi in range(1, NQ):
        cur = scores_for(qi)
        finish(qi - 1, *prev)
        prev = cur
    finish(NQ - 1, *prev)


def kernel(q, k, v, W, b):
    qf = q.reshape(L, H * D)
    kf = k.reshape(L, H * D)
    vf = v.reshape(L, H * D)

    scores = pl.pallas_call(
        _scores_kernel,
        grid=(1,),
        in_specs=[
            pl.BlockSpec((1, L, H, D), lambda i: (0, 0, 0, 0)),
            pl.BlockSpec((1, L, H, D), lambda i: (0, 0, 0, 0)),
        ],
        out_specs=pl.BlockSpec((H, NK, NQ), lambda i: (0, 0, 0)),
        out_shape=jax.ShapeDtypeStruct((H, NK, NQ), jnp.float32),
    )(q, k)

    sc_topk = functools.partial(
        pl.kernel,
        mesh=plsc.VectorSubcoreMesh(core_axis_name="c", subcore_axis_name="s"),
        out_type=jax.ShapeDtypeStruct((H, TOPK, NQ), jnp.int32),
        scratch_types=[
            pltpu.VMEM((NK, NQ), jnp.float32),
            pltpu.VMEM((TOPK, NQ), jnp.int32),
        ],
    )(_sc_topk_body)
    idx_full = sc_topk(scores)

    grid_spec = pltpu.PrefetchScalarGridSpec(
        num_scalar_prefetch=1,
        grid=(H,),
        in_specs=[
            pl.BlockSpec((L, D), lambda h, idx_ref: (0, h)),
            pl.BlockSpec((L, D), lambda h, idx_ref: (0, h)),
            pl.BlockSpec((L, D), lambda h, idx_ref: (0, h)),
        ],
        out_specs=pl.BlockSpec((L, D), lambda h, idx_ref: (0, h)),
    )
    o = pl.pallas_call(
        _attn_kernel,
        grid_spec=grid_spec,
        out_shape=jax.ShapeDtypeStruct((L, H * D), jnp.float32),
    )(idx_full, qf, kf, vf)

    return o.reshape(q.shape)
